# Initial kernel scaffold; baseline (speedup 1.0000x reference)
import jax
import jax.numpy as jnp
from jax.experimental import pallas as pl


def kernel(x):
    raise NotImplementedError("write your pallas kernel here")


# speedup vs baseline: 17.2005x; 17.2005x over previous
import functools

import jax
import jax.numpy as jnp
import numpy as np
from jax import lax
from jax.experimental import pallas as pl
from jax.experimental.pallas import tpu as pltpu
from jax.experimental.pallas import tpu_sc as plsc

_NUM_CLASSES = 1000
_SMOOTH = 0.1
_B = 16384
_L = 50

_HIT = np.float32(np.float32(1.0) * np.float32(1.0 - _SMOOTH)
                  + np.float32(_SMOOTH / _NUM_CLASSES))
_BG = np.float32(_SMOOTH / _NUM_CLASSES)

_NC = 2
_NS = 16
_NW = _NC * _NS
_ROWS_PER_W = _B // _NW
_CHUNK = 64
_NCHUNK = _ROWS_PER_W // _CHUNK
_IDX_PER_CHUNK = _CHUNK * _L
_VEC_ITERS = _IDX_PER_CHUNK // 16
_BUF_WORDS = _CHUNK * _NUM_CLASSES


def _body(x_hbm, out_hbm, idx_v, buf_v):
    cid = lax.axis_index("c")
    sid = lax.axis_index("s")
    wid = sid * _NC + cid

    hit = jnp.full((16,), _HIT, dtype=jnp.float32)
    bg = jnp.full((16,), _BG, dtype=jnp.float32)
    lanes = lax.iota(jnp.int32, 16)

    def init_step(i, _):
        buf_v[pl.ds(i * 16, 16)] = bg
        return 0
    lax.fori_loop(0, _BUF_WORDS // 16, init_step, 0)

    def scatter_pass(value_vec):
        def step(j, _):
            base = jnp.full((16,), j * 16, dtype=jnp.int32)
            p = base + lanes
            cls = idx_v[pl.ds(j * 16, 16)]
            row = lax.div(p, jnp.full((16,), _L, dtype=jnp.int32))
            tgt = row * jnp.full((16,), _NUM_CLASSES, dtype=jnp.int32) + cls
            plsc.store_scatter(buf_v, [tgt], value_vec)
            return 0
        lax.fori_loop(0, _VEC_ITERS, step, 0)

    for t in range(_NCHUNK):
        row_base = wid * _ROWS_PER_W + t * _CHUNK
        pltpu.sync_copy(x_hbm.at[pl.ds(row_base * _L, _IDX_PER_CHUNK)], idx_v)
        scatter_pass(hit)
        pltpu.sync_copy(buf_v, out_hbm.at[pl.ds(row_base * _NUM_CLASSES,
                                                _BUF_WORDS)])
        if t != _NCHUNK - 1:
            scatter_pass(bg)


@jax.jit
def _multihot(x_flat):
    mesh = plsc.VectorSubcoreMesh(core_axis_name="c", subcore_axis_name="s")
    fn = pl.kernel(
        _body,
        out_type=jax.ShapeDtypeStruct((_B * _NUM_CLASSES,), jnp.float32),
        mesh=mesh,
        scratch_types=[
            pltpu.VMEM((_IDX_PER_CHUNK,), jnp.int32),
            pltpu.VMEM((_BUF_WORDS,), jnp.float32),
        ],
        compiler_params=pltpu.CompilerParams(needs_layout_passes=False),
    )
    return fn(x_flat)


def kernel(x):
    x_flat = x.reshape(-1).astype(jnp.int32)
    out = _multihot(x_flat)
    return out.reshape(_B, _NUM_CLASSES)


# speedup vs baseline: 24.9046x; 1.4479x over previous
import jax
import jax.numpy as jnp
import numpy as np
from jax import lax
from jax.experimental import pallas as pl
from jax.experimental.pallas import tpu as pltpu
from jax.experimental.pallas import tpu_sc as plsc

_NUM_CLASSES = 1000
_SMOOTH = 0.1
_B = 16384
_L = 50

_HIT = np.float32(np.float32(1.0) * np.float32(1.0 - _SMOOTH)
                  + np.float32(_SMOOTH / _NUM_CLASSES))
_BG = np.float32(_SMOOTH / _NUM_CLASSES)

_NC = 2
_NS = 16
_NW = _NC * _NS
_ROWS_PER_W = _B // _NW
_CHUNK = 32
_NCHUNK = _ROWS_PER_W // _CHUNK
_COLS = (0, 16, 32, _L - 16)


def _body(x_hbm, out_hbm, idx_v0, idx_v1, buf_v0, buf_v1, sem0, sem1):
    cid = lax.axis_index("c")
    sid = lax.axis_index("s")
    wid = sid * _NC + cid

    idx_refs = (idx_v0, idx_v1)
    buf_refs = (buf_v0, buf_v1)
    sems = (sem0, sem1)

    hit = jnp.full((16,), _HIT, dtype=jnp.float32)
    bg = jnp.full((16,), _BG, dtype=jnp.float32)

    for b in range(2):
        buf = buf_refs[b]

        def init_step(r, _, buf=buf):
            def col_step(c, __):
                buf[r, pl.ds(c * 16, 16)] = bg
                return 0
            lax.fori_loop(0, _NUM_CLASSES // 16, col_step, 0)
            buf[r, pl.ds(_NUM_CLASSES - 16, 16)] = bg
            return 0
        lax.fori_loop(0, _CHUNK, init_step, 0)

    def scatter_pass(idx_ref, buf_ref, value_vec):
        def row_step(r, _):
            row_vec = jnp.full((16,), r, dtype=jnp.int32)
            for c in _COLS:
                cls = idx_ref[r, pl.ds(c, 16)]
                plsc.store_scatter(buf_ref, [row_vec, cls], value_vec)
            return 0
        lax.fori_loop(0, _CHUNK, row_step, 0)

    for t in range(_NCHUNK):
        s = t % 2
        idx_ref, buf_ref, sem = idx_refs[s], buf_refs[s], sems[s]
        row_base = wid * _ROWS_PER_W + t * _CHUNK
        if t >= 2:
            pltpu.make_async_copy(
                buf_ref, out_hbm.at[pl.ds(row_base, _CHUNK)], sem).wait()
            scatter_pass(idx_ref, buf_ref, bg)
        pltpu.sync_copy(x_hbm.at[pl.ds(row_base, _CHUNK)], idx_ref)
        scatter_pass(idx_ref, buf_ref, hit)
        pltpu.async_copy(buf_ref, out_hbm.at[pl.ds(row_base, _CHUNK)], sem)

    for t in (_NCHUNK - 2, _NCHUNK - 1):
        s = t % 2
        row_base = wid * _ROWS_PER_W + t * _CHUNK
        pltpu.make_async_copy(
            buf_refs[s], out_hbm.at[pl.ds(row_base, _CHUNK)], sems[s]).wait()


@jax.jit
def _multihot(x):
    mesh = plsc.VectorSubcoreMesh(core_axis_name="c", subcore_axis_name="s")
    fn = pl.kernel(
        _body,
        out_type=jax.ShapeDtypeStruct((_B, _NUM_CLASSES), jnp.float32),
        mesh=mesh,
        scratch_types=[
            pltpu.VMEM((_CHUNK, _L), jnp.int32),
            pltpu.VMEM((_CHUNK, _L), jnp.int32),
            pltpu.VMEM((_CHUNK, _NUM_CLASSES), jnp.float32),
            pltpu.VMEM((_CHUNK, _NUM_CLASSES), jnp.float32),
            pltpu.SemaphoreType.DMA,
            pltpu.SemaphoreType.DMA,
        ],
        compiler_params=pltpu.CompilerParams(needs_layout_passes=False),
    )
    return fn(x)


def kernel(x):
    return _multihot(x.astype(jnp.int32))
